# Initial kernel scaffold; baseline (speedup 1.0000x reference)
import jax
import jax.numpy as jnp
from jax.experimental import pallas as pl


def kernel(all_rois, gt_boxes, gt_labels, is_sample):
    raise NotImplementedError("write your pallas kernel here")


# speedup vs baseline: 3.6298x; 3.6298x over previous
import functools

import jax
import jax.numpy as jnp
from jax.experimental import pallas as pl
from jax.experimental.pallas import tpu as pltpu

FG_THRESHOLD = 0.5


def _tc_body(rois_ref, gtt_ref, gt_ref, labels_ref, gtrois_ref, *, nb, tn, m):
    lane = jax.lax.broadcasted_iota(jnp.int32, (tn, m), 1)
    for b in range(nb):
        g = gtt_ref[b]
        r = rois_ref[b]
        inter = None
        va = None
        vb = None
        for c in range(3):
            blo = r[:, 1 + c : 2 + c]
            bhi = r[:, 4 + c : 5 + c]
            glo = g[c : c + 1, :]
            ghi = g[3 + c : 4 + c, :]
            d = jnp.maximum(jnp.minimum(bhi, ghi) - jnp.maximum(blo, glo), 0.0)
            inter = d if inter is None else inter * d
            sa = jnp.maximum(bhi - blo, 0.0)
            va = sa if va is None else va * sa
            sb = jnp.maximum(ghi - glo, 0.0)
            vb = sb if vb is None else vb * sb
        union = jnp.maximum(va + vb - inter, 1e-9)
        iou = inter / union
        mx = jnp.max(iou, axis=1, keepdims=True)
        labels_ref[b] = (mx[:, 0] >= FG_THRESHOLD).astype(jnp.int32)
        sel = jnp.where(iou == mx, lane, m)
        amin = jnp.min(sel, axis=1, keepdims=True)
        onehot = (lane == amin).astype(jnp.float32)
        gtrois_ref[b] = jnp.dot(
            onehot, gt_ref[b], preferred_element_type=jnp.float32
        )


def kernel(all_rois, gt_boxes, gt_labels, is_sample):
    nb, n, _ = all_rois.shape
    m = gt_boxes.shape[1]
    tn = 512
    gt_t = jnp.swapaxes(gt_boxes, 1, 2)
    labels, gt_rois = pl.pallas_call(
        functools.partial(_tc_body, nb=nb, tn=tn, m=m),
        grid=(pl.cdiv(n, tn),),
        in_specs=[
            pl.BlockSpec((nb, tn, 7), lambda i: (0, i, 0)),
            pl.BlockSpec((nb, 6, m), lambda i: (0, 0, 0)),
            pl.BlockSpec((nb, m, 6), lambda i: (0, 0, 0)),
        ],
        out_specs=[
            pl.BlockSpec((nb, tn), lambda i: (0, i)),
            pl.BlockSpec((nb, tn, 6), lambda i: (0, i, 0)),
        ],
        out_shape=[
            jax.ShapeDtypeStruct((nb, n), jnp.int32),
            jax.ShapeDtypeStruct((nb, n, 6), jnp.float32),
        ],
        compiler_params=pltpu.CompilerParams(
            dimension_semantics=("arbitrary",),
        ),
    )(all_rois, gt_t, gt_boxes)
    return labels, all_rois, gt_rois
